# baseline (speedup 1.0000x reference)
import functools

import jax
import jax.numpy as jnp
from jax.experimental import pallas as pl
from jax.experimental.pallas import tpu as pltpu

_BLOCK = 32
_IN_F = 4096
_OUT_F = 4096
_GROUP = 128
_BLOCKS_PER_GROUP = _GROUP // _BLOCK
_N_ROW_GROUPS = _OUT_F // _GROUP
_N_COL_GROUPS = _IN_F // _GROUP
_TOKEN_TILE = 512


def _spmm_body(rg_ref, cg_ref, x_ref, w_ref, bias_ref, o_ref):
    b = pl.program_id(1)
    cg = cg_ref[b]
    xs = x_ref[:, pl.ds(cg * _GROUP, _GROUP)]
    contrib = jnp.dot(xs, w_ref[b], preferred_element_type=jnp.float32)
    prev_rg = rg_ref[jnp.maximum(b - 1, 0)]
    is_init = jnp.logical_or(b == 0, rg_ref[b] != prev_rg)

    @pl.when(is_init)
    def _():
        o_ref[...] = contrib + bias_ref[...]

    @pl.when(jnp.logical_not(is_init))
    def _():
        o_ref[...] += contrib


@jax.jit
def kernel(x, weight_data, block_rows, block_cols, bias):
    n_tokens = x.shape[0]
    n_blocks = weight_data.shape[0]

    rg = block_rows // _BLOCKS_PER_GROUP
    ro = block_rows % _BLOCKS_PER_GROUP
    cg = block_cols // _BLOCKS_PER_GROUP
    co = block_cols % _BLOCKS_PER_GROUP

    wt = jnp.transpose(weight_data, (0, 2, 1))
    w_tiles = jnp.zeros(
        (n_blocks, _BLOCKS_PER_GROUP, _BLOCK, _BLOCKS_PER_GROUP, _BLOCK),
        dtype=jnp.float32,
    )
    w_tiles = w_tiles.at[jnp.arange(n_blocks), co, :, ro, :].set(wt)
    w_tiles = w_tiles.reshape(n_blocks, _GROUP, _GROUP)

    rg_all = jnp.concatenate([rg, jnp.arange(_N_ROW_GROUPS, dtype=rg.dtype)])
    cg_all = jnp.concatenate(
        [cg, jnp.zeros((_N_ROW_GROUPS,), dtype=cg.dtype)])
    w_all = jnp.concatenate(
        [w_tiles, jnp.zeros((_N_ROW_GROUPS, _GROUP, _GROUP), jnp.float32)])

    order = jnp.argsort(rg_all, stable=True)
    rg_s = rg_all[order].astype(jnp.int32)
    cg_s = cg_all[order].astype(jnp.int32)
    w_s = w_all[order]

    bias2d = bias.reshape(1, _OUT_F)
    n_total = n_blocks + _N_ROW_GROUPS
    grid = (n_tokens // _TOKEN_TILE, n_total)

    grid_spec = pltpu.PrefetchScalarGridSpec(
        num_scalar_prefetch=2,
        grid=grid,
        in_specs=[
            pl.BlockSpec((_TOKEN_TILE, _IN_F), lambda t, b, rg, cg: (t, 0)),
            pl.BlockSpec((n_total, _GROUP, _GROUP),
                         lambda t, b, rg, cg: (0, 0, 0)),
            pl.BlockSpec((1, _GROUP), lambda t, b, rg, cg: (0, rg[b])),
        ],
        out_specs=pl.BlockSpec((_TOKEN_TILE, _GROUP),
                               lambda t, b, rg, cg: (t, rg[b])),
    )

    return pl.pallas_call(
        _spmm_body,
        grid_spec=grid_spec,
        out_shape=jax.ShapeDtypeStruct((n_tokens, _OUT_F), jnp.float32),
        compiler_params=pltpu.CompilerParams(
            dimension_semantics=("parallel", "arbitrary"),
        ),
    )(rg_s, cg_s, x, w_s, bias2d)


# speedup vs baseline: 1.9406x; 1.9406x over previous
import jax
import jax.numpy as jnp
from jax import lax
from jax.experimental import pallas as pl
from jax.experimental.pallas import tpu as pltpu

_BLOCK = 32
_IN_F = 4096
_OUT_F = 4096
_GROUP = 128
_BLOCKS_PER_GROUP = _GROUP // _BLOCK
_TOKEN_TILE = 512


def _spmm_body(rg_ref, cg_ref, x_ref, w_ref, bias_ref, o_ref):
    n_blocks = w_ref.shape[0]
    o_ref[...] = jnp.broadcast_to(bias_ref[...], o_ref.shape)

    def blk(b, carry):
        cg = cg_ref[b]
        rg = rg_ref[b]
        xs = x_ref[:, pl.ds(cg * _GROUP, _GROUP)]
        contrib = jnp.dot(xs, w_ref[b], preferred_element_type=jnp.float32)
        o_ref[:, pl.ds(rg * _GROUP, _GROUP)] += contrib
        return carry

    lax.fori_loop(0, n_blocks, blk, 0)


@jax.jit
def kernel(x, weight_data, block_rows, block_cols, bias):
    n_tokens = x.shape[0]
    n_blocks = weight_data.shape[0]

    rg = (block_rows // _BLOCKS_PER_GROUP).astype(jnp.int32)
    ro = block_rows % _BLOCKS_PER_GROUP
    cg = (block_cols // _BLOCKS_PER_GROUP).astype(jnp.int32)
    co = block_cols % _BLOCKS_PER_GROUP

    wt = jnp.transpose(weight_data, (0, 2, 1))
    w_tiles = jnp.zeros(
        (n_blocks, _BLOCKS_PER_GROUP, _BLOCK, _BLOCKS_PER_GROUP, _BLOCK),
        dtype=jnp.float32,
    )
    w_tiles = w_tiles.at[jnp.arange(n_blocks), co, :, ro, :].set(wt)
    w_tiles = w_tiles.reshape(n_blocks, _GROUP, _GROUP)

    bias2d = bias.reshape(1, _OUT_F)
    grid = (n_tokens // _TOKEN_TILE,)

    grid_spec = pltpu.PrefetchScalarGridSpec(
        num_scalar_prefetch=2,
        grid=grid,
        in_specs=[
            pl.BlockSpec((_TOKEN_TILE, _IN_F), lambda t, rg, cg: (t, 0)),
            pl.BlockSpec((n_blocks, _GROUP, _GROUP),
                         lambda t, rg, cg: (0, 0, 0)),
            pl.BlockSpec((1, _OUT_F), lambda t, rg, cg: (0, 0)),
        ],
        out_specs=pl.BlockSpec((_TOKEN_TILE, _OUT_F),
                               lambda t, rg, cg: (t, 0)),
    )

    return pl.pallas_call(
        _spmm_body,
        grid_spec=grid_spec,
        out_shape=jax.ShapeDtypeStruct((n_tokens, _OUT_F), jnp.float32),
        compiler_params=pltpu.CompilerParams(
            dimension_semantics=("arbitrary",),
        ),
    )(rg, cg, x, w_tiles, bias2d)


# speedup vs baseline: 3.1474x; 1.6218x over previous
import jax
import jax.numpy as jnp
from jax import lax
from jax.experimental import pallas as pl
from jax.experimental.pallas import tpu as pltpu

_BLOCK = 32
_IN_F = 4096
_OUT_F = 4096
_GROUP = 128
_BLOCKS_PER_GROUP = _GROUP // _BLOCK
_TOKEN_TILE = 512


def _spmm_body(rg_ref, cg_ref, x_ref, w_ref, bias_ref, o_ref):
    n_blocks = w_ref.shape[0]
    o_ref[...] = jnp.broadcast_to(bias_ref[...], o_ref.shape)

    def blk(b, carry):
        cg = cg_ref[b]
        rg = rg_ref[b]
        xs = x_ref[:, pl.ds(cg * _GROUP, _GROUP)]
        contrib = jnp.dot(xs, w_ref[b], preferred_element_type=jnp.float32)
        o_ref[:, pl.ds(rg * _GROUP, _GROUP)] += contrib
        return carry

    lax.fori_loop(0, n_blocks, blk, 0, unroll=4)


@jax.jit
def kernel(x, weight_data, block_rows, block_cols, bias):
    n_tokens = x.shape[0]
    n_blocks = weight_data.shape[0]

    rg = (block_rows // _BLOCKS_PER_GROUP).astype(jnp.int32)
    ro = block_rows % _BLOCKS_PER_GROUP
    cg = (block_cols // _BLOCKS_PER_GROUP).astype(jnp.int32)
    co = block_cols % _BLOCKS_PER_GROUP

    wt = jnp.transpose(weight_data, (0, 2, 1))
    slots = jnp.arange(_BLOCKS_PER_GROUP, dtype=jnp.int32)
    oh_co = (co[:, None] == slots).astype(jnp.float32)
    oh_ro = (ro[:, None] == slots).astype(jnp.float32)
    w_tiles = (wt[:, None, :, None, :]
               * oh_co[:, :, None, None, None]
               * oh_ro[:, None, None, :, None])
    w_tiles = w_tiles.reshape(n_blocks, _GROUP, _GROUP)

    bias2d = bias.reshape(1, _OUT_F)
    grid = (n_tokens // _TOKEN_TILE,)

    grid_spec = pltpu.PrefetchScalarGridSpec(
        num_scalar_prefetch=2,
        grid=grid,
        in_specs=[
            pl.BlockSpec((_TOKEN_TILE, _IN_F), lambda t, rg, cg: (t, 0)),
            pl.BlockSpec((n_blocks, _GROUP, _GROUP),
                         lambda t, rg, cg: (0, 0, 0)),
            pl.BlockSpec((1, _OUT_F), lambda t, rg, cg: (0, 0)),
        ],
        out_specs=pl.BlockSpec((_TOKEN_TILE, _OUT_F),
                               lambda t, rg, cg: (t, 0)),
    )

    return pl.pallas_call(
        _spmm_body,
        grid_spec=grid_spec,
        out_shape=jax.ShapeDtypeStruct((n_tokens, _OUT_F), jnp.float32),
        compiler_params=pltpu.CompilerParams(
            dimension_semantics=("arbitrary",),
        ),
    )(rg, cg, x, w_tiles, bias2d)


# speedup vs baseline: 3.6142x; 1.1483x over previous
import jax
import jax.numpy as jnp
from jax import lax
from jax.experimental import pallas as pl
from jax.experimental.pallas import tpu as pltpu

_BLOCK = 32
_IN_F = 4096
_OUT_F = 4096
_GROUP = 128
_BLOCKS_PER_GROUP = _GROUP // _BLOCK
_TOKEN_TILE = 512


def _spmm_body(rg_ref, cg_ref, x_ref, w_ref, bias_ref, o_ref, xb_ref):
    n_blocks = w_ref.shape[0]
    xb_ref[...] = x_ref[...].astype(jnp.bfloat16)
    o_ref[...] = jnp.broadcast_to(bias_ref[...], o_ref.shape)

    def blk(b, carry):
        cg = cg_ref[b]
        rg = rg_ref[b]
        xs = xb_ref[:, pl.ds(cg * _GROUP, _GROUP)]
        contrib = jnp.dot(xs, w_ref[b], preferred_element_type=jnp.float32)
        o_ref[:, pl.ds(rg * _GROUP, _GROUP)] += contrib
        return carry

    lax.fori_loop(0, n_blocks, blk, 0, unroll=8)


@jax.jit
def kernel(x, weight_data, block_rows, block_cols, bias):
    n_tokens = x.shape[0]
    n_blocks = weight_data.shape[0]

    rg = (block_rows // _BLOCKS_PER_GROUP).astype(jnp.int32)
    ro = block_rows % _BLOCKS_PER_GROUP
    cg = (block_cols // _BLOCKS_PER_GROUP).astype(jnp.int32)
    co = block_cols % _BLOCKS_PER_GROUP

    wt = jnp.transpose(weight_data, (0, 2, 1))
    slots = jnp.arange(_BLOCKS_PER_GROUP, dtype=jnp.int32)
    oh_co = (co[:, None] == slots).astype(jnp.float32)
    oh_ro = (ro[:, None] == slots).astype(jnp.float32)
    w_tiles = (wt[:, None, :, None, :]
               * oh_co[:, :, None, None, None]
               * oh_ro[:, None, None, :, None])
    w_tiles = w_tiles.reshape(n_blocks, _GROUP, _GROUP).astype(jnp.bfloat16)

    bias2d = bias.reshape(1, _OUT_F)
    grid = (n_tokens // _TOKEN_TILE,)

    grid_spec = pltpu.PrefetchScalarGridSpec(
        num_scalar_prefetch=2,
        grid=grid,
        in_specs=[
            pl.BlockSpec((_TOKEN_TILE, _IN_F), lambda t, rg, cg: (t, 0)),
            pl.BlockSpec((n_blocks, _GROUP, _GROUP),
                         lambda t, rg, cg: (0, 0, 0)),
            pl.BlockSpec((1, _OUT_F), lambda t, rg, cg: (0, 0)),
        ],
        out_specs=pl.BlockSpec((_TOKEN_TILE, _OUT_F),
                               lambda t, rg, cg: (t, 0)),
        scratch_shapes=[pltpu.VMEM((_TOKEN_TILE, _IN_F), jnp.bfloat16)],
    )

    return pl.pallas_call(
        _spmm_body,
        grid_spec=grid_spec,
        out_shape=jax.ShapeDtypeStruct((n_tokens, _OUT_F), jnp.float32),
        compiler_params=pltpu.CompilerParams(
            dimension_semantics=("arbitrary",),
        ),
    )(rg, cg, x, w_tiles, bias2d)
